# baseline (speedup 1.0000x reference)
import functools

import jax
import jax.numpy as jnp
from jax import lax
from jax.experimental import pallas as pl
from jax.experimental.pallas import tpu as pltpu
from jax.experimental.pallas import tpu_sc as plsc

F = 26
V = 100000
D = 32
B = 16384

NC = 2
NS = 16
NW = NC * NS
ROWS = B * F
ROWS_W = ROWS // NW
CHUNK = 1024
NCHUNK = ROWS_W // CHUNK
JPC = CHUNK // 128
VPC = CHUNK // 16


def _sc_gather(idx2d, flat_tab):
    mesh = plsc.VectorSubcoreMesh(core_axis_name="c", subcore_axis_name="s")

    @functools.partial(
        pl.kernel,
        mesh=mesh,
        out_type=jax.ShapeDtypeStruct((ROWS, D), jnp.float32),
        compiler_params=pltpu.CompilerParams(use_tc_tiling_on_sc=False),
        scratch_types=[
            pltpu.VMEM((2, JPC, 128), jnp.int32),
            pltpu.VMEM((2, CHUNK, D), jnp.float32),
            pltpu.SemaphoreType.DMA,
            pltpu.SemaphoreType.DMA,
            pltpu.SemaphoreType.DMA,
            pltpu.SemaphoreType.DMA,
            pltpu.SemaphoreType.DMA,
            pltpu.SemaphoreType.DMA,
        ],
    )
    def k(idx_hbm, tab_hbm, out_hbm, idx_v, rows_v,
          idx_s0, idx_s1, gat_s0, gat_s1, out_s0, out_s1):
        wid = lax.axis_index("s") * NC + lax.axis_index("c")
        irow0 = wid * (ROWS_W // 128)
        orow0 = wid * ROWS_W
        lane = lax.broadcasted_iota(jnp.int32, (16,), 0)

        idx_sems = (idx_s0, idx_s1)
        gat_sems = (gat_s0, gat_s1)
        out_sems = (out_s0, out_s1)

        def start_idx(c):
            b = c & 1
            return pltpu.async_copy(
                idx_hbm.at[pl.ds(irow0 + c * JPC, JPC)], idx_v.at[b],
                idx_sems[b])

        idx_cp = {0: start_idx(0)}
        out_cp = {}
        for c in range(NCHUNK):
            b = c & 1
            if c + 1 < NCHUNK:
                idx_cp[c + 1] = start_idx(c + 1)
            idx_cp[c].wait()

            def body(v, carry):
                j = v // 8
                col = (v % 8) * 16
                pos = c * CHUNK + v * 16 + lane
                off = (pos % F) * V
                idx_v[b, j, pl.ds(col, 16)] = (
                    idx_v[b, j, pl.ds(col, 16)] + off)
                return carry

            lax.fori_loop(0, VPC, body, 0)

            if c >= 2:
                out_cp[c - 2].wait()
            gats = [
                pltpu.async_copy(
                    tab_hbm.at[idx_v.at[b, j]],
                    rows_v.at[b, pl.ds(j * 128, 128)], gat_sems[b])
                for j in range(JPC)
            ]
            for g in gats:
                g.wait()
            out_cp[c] = pltpu.async_copy(
                rows_v.at[b], out_hbm.at[pl.ds(orow0 + c * CHUNK, CHUNK)],
                out_sems[b])
        out_cp[NCHUNK - 2].wait()
        out_cp[NCHUNK - 1].wait()

    return k(idx2d, flat_tab)


def kernel(x_cat, tables):
    idx2d = x_cat.reshape(ROWS // 128, 128)
    flat_tab = tables.reshape(F * V, D)
    out = _sc_gather(idx2d, flat_tab)
    return out.reshape(B, F * D)
